# baseline (speedup 1.0000x reference)
import functools

import jax
import jax.numpy as jnp
from jax import lax
from jax.experimental import pallas as pl
from jax.experimental.pallas import tpu as pltpu

N = 10000
ET = 3
NE = 160000
D = 256
F_RAW = 242
OUT = 16
L = 2
NEG = 0.01

BN = 1000


def _embed_body(inp_ref, e0_ref, e1_ref, e2_ref, out_ref):
    inp = inp_ref[...]
    parts = []
    for col, eref, v in ((0, e0_ref, 14), (1, e1_ref, 5), (2, e2_ref, 10)):
        c = inp[:, col].astype(jnp.int32)
        iot = lax.broadcasted_iota(jnp.int32, (BN, v), 1)
        oh = (iot == c[:, None]).astype(jnp.float32)
        parts.append(jnp.dot(oh, eref[...], preferred_element_type=jnp.float32))
    parts.append(inp[:, 3:])
    out_ref[...] = jnp.concatenate(parts, axis=1)


def _build_x(inputs, emb0, emb1, emb2):
    return pl.pallas_call(
        _embed_body,
        grid=(N // BN,),
        in_specs=[
            pl.BlockSpec((BN, F_RAW), lambda i: (i, 0)),
            pl.BlockSpec((14, 8), lambda i: (0, 0)),
            pl.BlockSpec((5, 3), lambda i: (0, 0)),
            pl.BlockSpec((10, 6), lambda i: (0, 0)),
        ],
        out_specs=pl.BlockSpec((BN, D), lambda i: (i, 0)),
        out_shape=jax.ShapeDtypeStruct((N, D), jnp.float32),
    )(inputs, emb0, emb1, emb2)


def _pool_body(feat_ref, w_ref, b_ref, out_ref, *, feat3d):
    f = feat_ref[0] if feat3d else feat_ref[...]
    acc = jnp.dot(f, w_ref[0], preferred_element_type=jnp.float32)
    out_ref[0] = jnp.maximum(acc + b_ref[0, 0], 0.0)


def _pooled(feat, W, b):
    feat3d = feat.ndim == 3
    fspec = (pl.BlockSpec((1, BN, D), lambda e, i: (e, i, 0)) if feat3d
             else pl.BlockSpec((BN, D), lambda e, i: (i, 0)))
    return pl.pallas_call(
        functools.partial(_pool_body, feat3d=feat3d),
        grid=(ET, N // BN),
        in_specs=[
            fspec,
            pl.BlockSpec((1, D, D), lambda e, i: (e, 0, 0)),
            pl.BlockSpec((1, 1, D), lambda e, i: (e, 0, 0)),
        ],
        out_specs=pl.BlockSpec((1, BN, D), lambda e, i: (e, i, 0)),
        out_shape=jax.ShapeDtypeStruct((ET, N, D), jnp.float32),
    )(feat, W, b)


def _sage_body(feat_ref, agg_ref, ws_ref, wn_ref, b_ref, out_ref, *, feat3d):
    f = feat_ref[0] if feat3d else feat_ref[...]
    acc = jnp.dot(f, ws_ref[0], preferred_element_type=jnp.float32)
    acc += jnp.dot(agg_ref[0], wn_ref[0], preferred_element_type=jnp.float32)
    acc += b_ref[0, 0]
    out_ref[0] = jnp.where(acc >= 0.0, acc, NEG * acc)


def _sage_out(feat, agg, Ws, Wn, b):
    feat3d = feat.ndim == 3
    fspec = (pl.BlockSpec((1, BN, D), lambda e, i: (e, i, 0)) if feat3d
             else pl.BlockSpec((BN, D), lambda e, i: (i, 0)))
    return pl.pallas_call(
        functools.partial(_sage_body, feat3d=feat3d),
        grid=(ET, N // BN),
        in_specs=[
            fspec,
            pl.BlockSpec((1, BN, D), lambda e, i: (e, i, 0)),
            pl.BlockSpec((1, D, D), lambda e, i: (e, 0, 0)),
            pl.BlockSpec((1, D, D), lambda e, i: (e, 0, 0)),
            pl.BlockSpec((1, 1, D), lambda e, i: (e, 0, 0)),
        ],
        out_specs=pl.BlockSpec((1, BN, D), lambda e, i: (e, i, 0)),
        out_shape=jax.ShapeDtypeStruct((ET, N, D), jnp.float32),
    )(feat, agg, Ws, Wn, b)


def _att_body(h_ref, a_ref, wl_ref, bl_ref, wo_ref, bo_ref, out_ref):
    h = [h_ref[e] for e in range(ET)]
    A = a_ref[...]
    y = [jnp.dot(h[e], A, preferred_element_type=jnp.float32) for e in range(ET)]
    s = [[jnp.sum(y[f] * h[e], axis=1, keepdims=True) for e in range(ET)]
         for f in range(ET)]
    att = []
    for e in range(ET):
        m = jnp.maximum(jnp.maximum(s[0][e], s[1][e]), s[2][e])
        w = [jnp.exp(s[f][e] - m) for f in range(ET)]
        z = w[0] + w[1] + w[2]
        att.append((w[0] * h[0] + w[1] * h[1] + w[2] * h[2]) / z)
    hc = jnp.concatenate(att, axis=1)
    hid = jnp.dot(hc, wl_ref[...], preferred_element_type=jnp.float32) + bl_ref[...]
    hid = jnp.where(hid >= 0.0, hid, NEG * hid)
    out_ref[...] = jnp.dot(hid, wo_ref[...], preferred_element_type=jnp.float32) + bo_ref[...]


def _attention_head(h, att_mat, W_lin, b_lin, W_out, b_out):
    return pl.pallas_call(
        _att_body,
        grid=(N // BN,),
        in_specs=[
            pl.BlockSpec((ET, BN, D), lambda i: (0, i, 0)),
            pl.BlockSpec((D, D), lambda i: (0, 0)),
            pl.BlockSpec((ET * D, D), lambda i: (0, 0)),
            pl.BlockSpec((D,), lambda i: (0,)),
            pl.BlockSpec((D, OUT), lambda i: (0, 0)),
            pl.BlockSpec((OUT,), lambda i: (0,)),
        ],
        out_specs=pl.BlockSpec((BN, OUT), lambda i: (i, 0)),
        out_shape=jax.ShapeDtypeStruct((N, OUT), jnp.float32),
    )(h, att_mat, W_lin, b_lin, W_out, b_out)


def _segment_max(pooled, src, dst):
    outs = []
    for e in range(ET):
        gath = pooled[e][src[e]]
        seg = jax.ops.segment_max(gath, dst[e], num_segments=N)
        outs.append(jnp.where(jnp.isfinite(seg), seg, 0.0))
    return jnp.stack(outs)


def kernel(inputs, edge_index, emb0, emb1, emb2, W_pool, b_pool, W_self,
           W_neigh, b_sage, att_mat, W_lin, b_lin, W_out, b_out):
    src = edge_index[:, 0]
    dst = edge_index[:, 1]

    x = _build_x(inputs, emb0, emb1, emb2)

    feat = x
    for l in range(L):
        pooled = _pooled(feat, W_pool[l], b_pool[l].reshape(ET, 1, D))
        agg = _segment_max(pooled, src, dst)
        feat = _sage_out(feat, agg, W_self[l], W_neigh[l], b_sage[l].reshape(ET, 1, D))

    return _attention_head(feat, att_mat, W_lin, b_lin, W_out, b_out)


# speedup vs baseline: 1.4936x; 1.4936x over previous
import functools

import jax
import jax.numpy as jnp
from jax import lax
from jax.experimental import pallas as pl
from jax.experimental.pallas import tpu as pltpu
from jax.experimental.pallas import tpu_sc as plsc

N = 10000
ET = 3
NE = 160000
D = 256
F_RAW = 242
OUT = 16
L = 2
NEG = 0.01

BN = 1000


def _embed_body(inp_ref, e0_ref, e1_ref, e2_ref, out_ref):
    inp = inp_ref[...]
    parts = []
    for col, eref, v in ((0, e0_ref, 14), (1, e1_ref, 5), (2, e2_ref, 10)):
        c = inp[:, col].astype(jnp.int32)
        iot = lax.broadcasted_iota(jnp.int32, (BN, v), 1)
        oh = (iot == c[:, None]).astype(jnp.float32)
        parts.append(jnp.dot(oh, eref[...], preferred_element_type=jnp.float32))
    parts.append(inp[:, 3:])
    out_ref[...] = jnp.concatenate(parts, axis=1)


def _build_x(inputs, emb0, emb1, emb2):
    return pl.pallas_call(
        _embed_body,
        grid=(N // BN,),
        in_specs=[
            pl.BlockSpec((BN, F_RAW), lambda i: (i, 0)),
            pl.BlockSpec((14, 8), lambda i: (0, 0)),
            pl.BlockSpec((5, 3), lambda i: (0, 0)),
            pl.BlockSpec((10, 6), lambda i: (0, 0)),
        ],
        out_specs=pl.BlockSpec((BN, D), lambda i: (i, 0)),
        out_shape=jax.ShapeDtypeStruct((N, D), jnp.float32),
    )(inputs, emb0, emb1, emb2)


def _pool_body(feat_ref, w_ref, b_ref, out_ref, *, feat3d):
    f = feat_ref[0] if feat3d else feat_ref[...]
    acc = jnp.dot(f, w_ref[0], preferred_element_type=jnp.float32)
    out_ref[0] = jnp.maximum(acc + b_ref[0, 0], 0.0)


def _pooled(feat, W, b):
    feat3d = feat.ndim == 3
    fspec = (pl.BlockSpec((1, BN, D), lambda e, i: (e, i, 0)) if feat3d
             else pl.BlockSpec((BN, D), lambda e, i: (i, 0)))
    return pl.pallas_call(
        functools.partial(_pool_body, feat3d=feat3d),
        grid=(ET, N // BN),
        in_specs=[
            fspec,
            pl.BlockSpec((1, D, D), lambda e, i: (e, 0, 0)),
            pl.BlockSpec((1, 1, D), lambda e, i: (e, 0, 0)),
        ],
        out_specs=pl.BlockSpec((1, BN, D), lambda e, i: (e, i, 0)),
        out_shape=jax.ShapeDtypeStruct((ET, N, D), jnp.float32),
    )(feat, W, b)


def _sage_body(feat_ref, agg_ref, ws_ref, wn_ref, b_ref, out_ref, *, feat3d):
    f = feat_ref[0] if feat3d else feat_ref[...]
    acc = jnp.dot(f, ws_ref[0], preferred_element_type=jnp.float32)
    acc += jnp.dot(agg_ref[0], wn_ref[0], preferred_element_type=jnp.float32)
    acc += b_ref[0, 0]
    out_ref[0] = jnp.where(acc >= 0.0, acc, NEG * acc)


def _sage_out(feat, agg, Ws, Wn, b):
    feat3d = feat.ndim == 3
    fspec = (pl.BlockSpec((1, BN, D), lambda e, i: (e, i, 0)) if feat3d
             else pl.BlockSpec((BN, D), lambda e, i: (i, 0)))
    return pl.pallas_call(
        functools.partial(_sage_body, feat3d=feat3d),
        grid=(ET, N // BN),
        in_specs=[
            fspec,
            pl.BlockSpec((1, BN, D), lambda e, i: (e, i, 0)),
            pl.BlockSpec((1, D, D), lambda e, i: (e, 0, 0)),
            pl.BlockSpec((1, D, D), lambda e, i: (e, 0, 0)),
            pl.BlockSpec((1, 1, D), lambda e, i: (e, 0, 0)),
        ],
        out_specs=pl.BlockSpec((1, BN, D), lambda e, i: (e, i, 0)),
        out_shape=jax.ShapeDtypeStruct((ET, N, D), jnp.float32),
    )(feat, agg, Ws, Wn, b)


def _att_body(h_ref, a_ref, wl_ref, bl_ref, wo_ref, bo_ref, out_ref):
    h = [h_ref[e] for e in range(ET)]
    A = a_ref[...]
    y = [jnp.dot(h[e], A, preferred_element_type=jnp.float32) for e in range(ET)]
    s = [[jnp.sum(y[f] * h[e], axis=1, keepdims=True) for e in range(ET)]
         for f in range(ET)]
    att = []
    for e in range(ET):
        m = jnp.maximum(jnp.maximum(s[0][e], s[1][e]), s[2][e])
        w = [jnp.exp(s[f][e] - m) for f in range(ET)]
        z = w[0] + w[1] + w[2]
        att.append((w[0] * h[0] + w[1] * h[1] + w[2] * h[2]) / z)
    hc = jnp.concatenate(att, axis=1)
    hid = jnp.dot(hc, wl_ref[...], preferred_element_type=jnp.float32) + bl_ref[...]
    hid = jnp.where(hid >= 0.0, hid, NEG * hid)
    out_ref[...] = jnp.dot(hid, wo_ref[...], preferred_element_type=jnp.float32) + bo_ref[...]


def _attention_head(h, att_mat, W_lin, b_lin, W_out, b_out):
    return pl.pallas_call(
        _att_body,
        grid=(N // BN,),
        in_specs=[
            pl.BlockSpec((ET, BN, D), lambda i: (0, i, 0)),
            pl.BlockSpec((D, D), lambda i: (0, 0)),
            pl.BlockSpec((ET * D, D), lambda i: (0, 0)),
            pl.BlockSpec((D,), lambda i: (0,)),
            pl.BlockSpec((D, OUT), lambda i: (0, 0)),
            pl.BlockSpec((OUT,), lambda i: (0,)),
        ],
        out_specs=pl.BlockSpec((BN, OUT), lambda i: (i, 0)),
        out_shape=jax.ShapeDtypeStruct((N, OUT), jnp.float32),
    )(h, att_mat, W_lin, b_lin, W_out, b_out)


SC_NC = 2
SC_NS = 16
SC_NW = SC_NC * SC_NS
NPT = 320
NP = SC_NW * NPT
EC = 2000
NCHUNK = NE // EC
NPAIR = NCHUNK // 2
SB = 2048
STRASH = SB + 16
LC = 2048
GB = 64
NEP = NCHUNK * LC
LDSH = 15
DUMMY_PACK = NPT << LDSH


def _build_body(src_ref, dst_ref, lists_ref, counts_ref,
                stag0, stag1, dv0, sv0, dv1, sv1,
                sem_f0, sem_f1, sem_e0, sem_e1):
    wid = lax.axis_index("c") * SC_NS + lax.axis_index("s")
    base = wid * NPT
    dummy16 = jnp.full((16,), DUMMY_PACK, jnp.int32)

    def wait_edge(dv, sv, sem, ch):
        off = pl.multiple_of(ch, 8)
        pltpu.make_async_copy(dst_ref.at[pl.ds(off, EC)], dv, sem).wait()
        pltpu.make_async_copy(src_ref.at[pl.ds(off, EC)], sv, sem).wait()

    for e in range(ET):
        rbase = (e * SC_NW + wid) * NEP
        ebase = e * NE

        def scan_chunk(stag, dv, sv):
            def g_body(g, wptr):
                d16 = dv[pl.ds(g * 16, 16)]
                s16 = sv[pl.ds(g * 16, 16)]
                m = (d16 >= base) & (d16 < base + NPT)
                cs = plsc.cumsum(m.astype(jnp.int32))
                pos = jnp.where(m, wptr - 1 + cs, STRASH)
                pack = ((d16 - base) << LDSH) | (s16 + e * N)
                plsc.store_scatter(stag, [pos], pack)
                return wptr + cs[15]
            cnt = lax.fori_loop(0, EC // 16, g_body, 0, unroll=2)
            stag[pl.ds(cnt, 16)] = dummy16
            return (cnt + 7) & ~7

        pltpu.sync_copy(dst_ref.at[pl.ds(ebase, EC)], dv0)
        pltpu.sync_copy(src_ref.at[pl.ds(ebase, EC)], sv0)
        pltpu.async_copy(dst_ref.at[pl.ds(ebase + EC, EC)], dv1, sem_e1)
        pltpu.async_copy(src_ref.at[pl.ds(ebase + EC, EC)], sv1, sem_e1)

        def pair_body(p, qoff):
            ch0 = 2 * p

            @pl.when(p > 0)
            def _():
                wait_edge(dv0, sv0, sem_e0, ebase + ch0 * EC)
                pltpu.make_async_copy(
                    stag0.at[pl.ds(0, SB)],
                    lists_ref.at[pl.ds(pl.multiple_of(rbase, 8), SB)], sem_f0).wait()

            pad0 = scan_chunk(stag0, dv0, sv0)
            pltpu.async_copy(stag0.at[pl.ds(0, SB)],
                             lists_ref.at[pl.ds(pl.multiple_of(rbase + qoff, 8), SB)], sem_f0)
            qoff1 = qoff + pad0

            @pl.when(p < NPAIR - 1)
            def _():
                nxt = pl.multiple_of(ebase + (ch0 + 2) * EC, 8)
                pltpu.async_copy(dst_ref.at[pl.ds(nxt, EC)], dv0, sem_e0)
                pltpu.async_copy(src_ref.at[pl.ds(nxt, EC)], sv0, sem_e0)

            wait_edge(dv1, sv1, sem_e1, ebase + (ch0 + 1) * EC)

            @pl.when(p > 0)
            def _():
                pltpu.make_async_copy(
                    stag1.at[pl.ds(0, SB)],
                    lists_ref.at[pl.ds(pl.multiple_of(rbase, 8), SB)], sem_f1).wait()

            pad1 = scan_chunk(stag1, dv1, sv1)
            pltpu.async_copy(stag1.at[pl.ds(0, SB)],
                             lists_ref.at[pl.ds(pl.multiple_of(rbase + qoff1, 8), SB)], sem_f1)
            qoff2 = qoff1 + pad1

            @pl.when(p < NPAIR - 1)
            def _():
                nxt = pl.multiple_of(ebase + (ch0 + 3) * EC, 8)
                pltpu.async_copy(dst_ref.at[pl.ds(nxt, EC)], dv1, sem_e1)
                pltpu.async_copy(src_ref.at[pl.ds(nxt, EC)], sv1, sem_e1)

            return qoff2

        q_final = lax.fori_loop(0, NPAIR, pair_body, 0)

        pltpu.make_async_copy(stag0.at[pl.ds(0, SB)],
                              lists_ref.at[pl.ds(pl.multiple_of(rbase, 8), SB)], sem_f0).wait()
        pltpu.make_async_copy(stag1.at[pl.ds(0, SB)],
                              lists_ref.at[pl.ds(pl.multiple_of(rbase, 8), SB)], sem_f1).wait()

        for t in range(GB // 16):
            stag0[pl.ds(t * 16, 16)] = dummy16
        pltpu.sync_copy(stag0.at[pl.ds(0, GB)],
                        lists_ref.at[pl.ds(pl.multiple_of(rbase + q_final, 8), GB)])
        lane = lax.broadcasted_iota(jnp.int32, (16,), 0)
        stag1[pl.ds(0, 16)] = jnp.where(lane == 0, q_final, 0)
        pltpu.sync_copy(stag1.at[pl.ds(0, 16)],
                        counts_ref.at[pl.ds(pl.multiple_of((e * SC_NW + wid) * 16, 8), 16)])


def _agg_body(pooled_ref, lists_ref, counts_ref, out_ref,
              agg, lbuf, cv, idx0, idx1, rows0, rows1, sem_g0, sem_g1):
    wid = lax.axis_index("c") * SC_NS + lax.axis_index("s")
    base = wid * NPT
    zero16f = jnp.zeros((16,), jnp.float32)
    pltpu.sync_copy(counts_ref, cv)

    def unpack(b, idxbuf):
        for g2 in range(GB // 16):
            pk = lbuf[pl.ds(b * GB + g2 * 16, 16)]
            idxbuf[pl.ds(g2 * 16, 16)] = pk & ((1 << LDSH) - 1)

    def start(idxbuf, rowsx, semx):
        pltpu.async_copy(pooled_ref.at[idxbuf], rowsx, semx)

    def wait(idxbuf, rowsx, semx):
        pltpu.make_async_copy(pooled_ref.at[idxbuf], rowsx, semx).wait()

    def update(b, rowsx):
        def jbody(j, _):
            pk = lbuf[pl.ds(b * GB + j, 16)][0]
            ld = pk >> LDSH
            for c in range(D // 16):
                sl = pl.ds(c * 16, 16)
                agg[ld, sl] = jnp.maximum(agg[ld, sl], rowsx[j, sl])
            return 0
        lax.fori_loop(0, GB, jbody, 0, unroll=2)

    for e in range(ET):
        def zrow(r, _):
            for c in range(D // 16):
                agg[r, pl.ds(c * 16, 16)] = zero16f
            return 0
        lax.fori_loop(0, NPT + 1, zrow, 0, unroll=2)

        q_len = cv[pl.ds(pl.multiple_of((e * SC_NW + wid) * 16, 8), 16)][0]
        rbase = (e * SC_NW + wid) * NEP

        def lc_body(lc, _):
            pltpu.sync_copy(lists_ref.at[pl.ds(pl.multiple_of(rbase + lc * LC, 8), LC)],
                            lbuf.at[pl.ds(0, LC)])
            rem = q_len - lc * LC
            nb = jnp.minimum((rem + GB - 1) // GB, LC // GB)

            @pl.when(nb > 0)
            def _():
                unpack(0, idx0)
                start(idx0, rows0, sem_g0)

            def pair(pp, _):
                b0 = 2 * pp

                @pl.when(b0 + 1 < nb)
                def _():
                    unpack(b0 + 1, idx1)
                    start(idx1, rows1, sem_g1)

                @pl.when(b0 < nb)
                def _():
                    wait(idx0, rows0, sem_g0)
                    update(b0, rows0)

                @pl.when(b0 + 2 < nb)
                def _():
                    unpack(b0 + 2, idx0)
                    start(idx0, rows0, sem_g0)

                @pl.when(b0 + 1 < nb)
                def _():
                    wait(idx1, rows1, sem_g1)
                    update(b0 + 1, rows1)

                return 0

            lax.fori_loop(0, LC // GB // 2, pair, 0)
            return 0

        lax.fori_loop(0, (q_len + LC - 1) // LC, lc_body, 0)
        pltpu.sync_copy(agg.at[pl.ds(0, NPT)],
                        out_ref.at[e, pl.ds(pl.multiple_of(base, 8), NPT)])


def _sc_mesh():
    return plsc.VectorSubcoreMesh(core_axis_name="c", subcore_axis_name="s",
                                  num_cores=SC_NC, num_subcores=SC_NS)


def _build_lists(src, dst):
    k = functools.partial(
        pl.kernel,
        out_type=(jax.ShapeDtypeStruct((ET * SC_NW * NEP,), jnp.int32),
                  jax.ShapeDtypeStruct((ET * SC_NW * 16,), jnp.int32)),
        mesh=_sc_mesh(),
        compiler_params=pltpu.CompilerParams(needs_layout_passes=False),
        scratch_types=[
            pltpu.VMEM((SB + 32,), jnp.int32),
            pltpu.VMEM((SB + 32,), jnp.int32),
            pltpu.VMEM((EC,), jnp.int32),
            pltpu.VMEM((EC,), jnp.int32),
            pltpu.VMEM((EC,), jnp.int32),
            pltpu.VMEM((EC,), jnp.int32),
            pltpu.SemaphoreType.DMA,
            pltpu.SemaphoreType.DMA,
            pltpu.SemaphoreType.DMA,
            pltpu.SemaphoreType.DMA,
        ],
    )(_build_body)
    return k(src, dst)


def _aggregate(pooled, lists, counts):
    k = functools.partial(
        pl.kernel,
        out_type=jax.ShapeDtypeStruct((ET, NP, D), jnp.float32),
        mesh=_sc_mesh(),
        compiler_params=pltpu.CompilerParams(needs_layout_passes=False),
        scratch_types=[
            pltpu.VMEM((NPT + 1, D), jnp.float32),
            pltpu.VMEM((LC + 16,), jnp.int32),
            pltpu.VMEM((ET * SC_NW * 16,), jnp.int32),
            pltpu.VMEM((GB,), jnp.int32),
            pltpu.VMEM((GB,), jnp.int32),
            pltpu.VMEM((GB, D), jnp.float32),
            pltpu.VMEM((GB, D), jnp.float32),
            pltpu.SemaphoreType.DMA,
            pltpu.SemaphoreType.DMA,
        ],
    )(_agg_body)
    out = k(pooled.reshape(ET * N, D), lists, counts)
    return out[:, :N]


def kernel(inputs, edge_index, emb0, emb1, emb2, W_pool, b_pool, W_self,
           W_neigh, b_sage, att_mat, W_lin, b_lin, W_out, b_out):
    src = edge_index[:, 0].reshape(ET * NE)
    dst = edge_index[:, 1].reshape(ET * NE)

    x = _build_x(inputs, emb0, emb1, emb2)
    lists, counts = _build_lists(src, dst)

    feat = x
    for l in range(L):
        pooled = _pooled(feat, W_pool[l], b_pool[l].reshape(ET, 1, D))
        agg = _aggregate(pooled, lists, counts)
        feat = _sage_out(feat, agg, W_self[l], W_neigh[l], b_sage[l].reshape(ET, 1, D))

    return _attention_head(feat, att_mat, W_lin, b_lin, W_out, b_out)
